# baseline (speedup 1.0000x reference)
import functools

import jax
import jax.numpy as jnp
from jax import lax
from jax.experimental import pallas as pl
from jax.experimental.pallas import tpu as pltpu
from jax.experimental.pallas import tpu_sc as plsc

EMB = 128
V0, V1, V2 = 4, 2, 6
NCOMBO = V0 * V1 * V2
E = 320000
NC, NS = 2, 16
NW = NC * NS
SLAB = 128
NSLAB = E // SLAB
BASE = NSLAB // NW
EXTRA = NSLAB % NW
MAXSLABS = BASE + 1
K = 4
LAG = 3


def _bond_kernel(e0_hbm, e1_hbm, e2_hbm, w0_hbm, w1_hbm, w2_hbm, out_hbm,
                 t_hbm, e0_v, e1_v, e2_v, pk_v, rows_v, w0_v, w1_v, w2_v,
                 t_v, gsem, ssem):
    cid = lax.axis_index("c")
    sid = lax.axis_index("s")
    wid = sid * NC + cid

    @pl.when(sid == 0)
    def _build_table():
        pltpu.sync_copy(w0_hbm, w0_v)
        pltpu.sync_copy(w1_hbm, w1_v)
        pltpu.sync_copy(w2_hbm, w2_v)
        for r in range(NCOMBO):
            i, j, k = r // (V1 * V2), (r // V2) % V1, r % V2

            def _g(g, carry, i=i, j=j, k=k, r=r):
                sl = pl.ds(g * 16, 16)
                t_v[r, sl] = w0_v[i, sl] + w1_v[j, sl] + w2_v[k, sl]
                return carry

            lax.fori_loop(0, EMB // 16, _g, 0)
        pltpu.sync_copy(t_v, t_hbm.at[pl.ds(cid * NCOMBO, NCOMBO)])

    plsc.subcore_barrier()

    start_slab = wid * BASE + jnp.maximum(wid - (NW - EXTRA), 0)
    ea_base = start_slab * SLAB
    pltpu.sync_copy(e0_hbm.at[pl.ds(ea_base, MAXSLABS * SLAB)], e0_v)
    pltpu.sync_copy(e1_hbm.at[pl.ds(ea_base, MAXSLABS * SLAB)], e1_v)
    pltpu.sync_copy(e2_hbm.at[pl.ds(ea_base, MAXSLABS * SLAB)], e2_v)

    tbase = (cid * NCOMBO).astype(jnp.int32)

    def _pk(t, carry):
        sl = pl.ds(t * 16, 16)
        pk_v[sl] = e0_v[sl] * (V1 * V2) + e1_v[sl] * V2 + e2_v[sl] + tbase
        return carry

    lax.fori_loop(0, MAXSLABS * SLAB // 16, _pk, 0)

    nslabs = jnp.where(wid >= NW - EXTRA, MAXSLABS, BASE)

    def _gwait(slot):
        pltpu.make_async_copy(t_hbm.at[pk_v.at[pl.ds(0, SLAB)]],
                              rows_v.at[slot], gsem.at[slot]).wait()

    def _swait(slot):
        pltpu.make_async_copy(rows_v.at[slot], out_hbm.at[pl.ds(0, SLAB)],
                              ssem.at[slot]).wait()

    def _body(t, carry):
        slot = lax.rem(t, K)

        @pl.when(t < nslabs)
        def _start_gather():
            @pl.when(t >= K)
            def _buffer_free():
                _swait(slot)

            idx = pk_v.at[pl.ds(t * SLAB, SLAB)]
            pltpu.async_copy(t_hbm.at[idx], rows_v.at[slot], gsem.at[slot])

        u = t - LAG

        @pl.when(u >= 0)
        def _start_scatter():
            uslot = lax.rem(u, K)
            _gwait(uslot)
            pltpu.async_copy(rows_v.at[uslot],
                             out_hbm.at[pl.ds((start_slab + u) * SLAB, SLAB)],
                             ssem.at[uslot])

        return carry

    lax.fori_loop(0, nslabs + LAG, _body, 0)
    for s in range(K):
        _swait(s)


@jax.jit
def _run(e0, e1, e2, w0, w1, w2):
    mesh = plsc.VectorSubcoreMesh(core_axis_name="c", subcore_axis_name="s",
                                  num_cores=NC, num_subcores=NS)
    out, _ = pl.kernel(
        _bond_kernel,
        out_type=(
            jax.ShapeDtypeStruct((E, EMB), jnp.float32),
            jax.ShapeDtypeStruct((NC * NCOMBO, EMB), jnp.float32),
        ),
        mesh=mesh,
        scratch_types=[
            pltpu.VMEM((MAXSLABS * SLAB,), jnp.int32),
            pltpu.VMEM((MAXSLABS * SLAB,), jnp.int32),
            pltpu.VMEM((MAXSLABS * SLAB,), jnp.int32),
            pltpu.VMEM((MAXSLABS * SLAB,), jnp.int32),
            pltpu.VMEM((K, SLAB, EMB), jnp.float32),
            pltpu.VMEM((V0, EMB), jnp.float32),
            pltpu.VMEM((V1, EMB), jnp.float32),
            pltpu.VMEM((V2, EMB), jnp.float32),
            pltpu.VMEM((NCOMBO, EMB), jnp.float32),
            pltpu.SemaphoreType.DMA((K,)),
            pltpu.SemaphoreType.DMA((K,)),
        ],
    )(e0, e1, e2, w0, w1, w2)
    return out


def kernel(edge_attr, W0, W1, W2):
    ea = edge_attr.astype(jnp.int32)
    return _run(ea[:, 0], ea[:, 1], ea[:, 2], W0, W1, W2)


# speedup vs baseline: 5.0667x; 5.0667x over previous
import jax
import jax.numpy as jnp
from jax import lax
from jax.experimental import pallas as pl
from jax.experimental.pallas import tpu as pltpu
from jax.experimental.pallas import tpu_sc as plsc

EMB = 128
V0, V1, V2 = 4, 2, 6
NCOMBO = V0 * V1 * V2
E = 320000
NC, NS = 2, 16
NW = NC * NS
SLAB = 128
NSLAB = E // SLAB
BASE = NSLAB // NW
EXTRA = NSLAB % NW
MAXSLABS = BASE + 1
K = 4


def _bond_kernel(e0_hbm, e1_hbm, e2_hbm, w0_hbm, w1_hbm, w2_hbm, out_hbm,
                 e0_v, e1_v, e2_v, pk_v, rows_v, w0_v, w1_v, w2_v,
                 t_v, ssem):
    cid = lax.axis_index("c")
    sid = lax.axis_index("s")
    wid = sid * NC + cid

    pltpu.sync_copy(w0_hbm, w0_v)
    pltpu.sync_copy(w1_hbm, w1_v)
    pltpu.sync_copy(w2_hbm, w2_v)
    for r in range(NCOMBO):
        i, j, k = r // (V1 * V2), (r // V2) % V1, r % V2

        def _g(g, carry, i=i, j=j, k=k, r=r):
            sl = pl.ds(g * 16, 16)
            t_v[r, sl] = w0_v[i, sl] + w1_v[j, sl] + w2_v[k, sl]
            return carry

        lax.fori_loop(0, EMB // 16, _g, 0)

    start_slab = wid * BASE + jnp.maximum(wid - (NW - EXTRA), 0)
    ea_base = start_slab * SLAB
    pltpu.sync_copy(e0_hbm.at[pl.ds(ea_base, MAXSLABS * SLAB)], e0_v)
    pltpu.sync_copy(e1_hbm.at[pl.ds(ea_base, MAXSLABS * SLAB)], e1_v)
    pltpu.sync_copy(e2_hbm.at[pl.ds(ea_base, MAXSLABS * SLAB)], e2_v)

    def _pk(t, carry):
        sl = pl.ds(t * 16, 16)
        pk_v[sl] = e0_v[sl] * (V1 * V2) + e1_v[sl] * V2 + e2_v[sl]
        return carry

    lax.fori_loop(0, MAXSLABS * SLAB // 16, _pk, 0)

    nslabs = jnp.where(wid >= NW - EXTRA, MAXSLABS, BASE)

    def _swait(slot):
        pltpu.make_async_copy(rows_v.at[slot], out_hbm.at[pl.ds(0, SLAB)],
                              ssem.at[slot]).wait()

    def _slab(t, carry):
        slot = lax.rem(t, K)

        @pl.when(t >= K)
        def _buffer_free():
            _swait(slot)

        def _grp(q, c2):
            pk16 = pk_v[pl.ds(t * SLAB + q * 16, 16)]
            for li in range(16):
                r = pk16[li]
                e = q * 16 + li
                for g in range(EMB // 16):
                    sl = pl.ds(g * 16, 16)
                    rows_v[slot, e, sl] = t_v[r, sl]
            return c2

        lax.fori_loop(0, SLAB // 16, _grp, 0)
        pltpu.async_copy(rows_v.at[slot],
                         out_hbm.at[pl.ds((start_slab + t) * SLAB, SLAB)],
                         ssem.at[slot])
        return carry

    lax.fori_loop(0, nslabs, _slab, 0)
    for s in range(K):
        _swait(s)


@jax.jit
def _run(e0, e1, e2, w0, w1, w2):
    mesh = plsc.VectorSubcoreMesh(core_axis_name="c", subcore_axis_name="s",
                                  num_cores=NC, num_subcores=NS)
    out = pl.kernel(
        _bond_kernel,
        out_type=jax.ShapeDtypeStruct((E, EMB), jnp.float32),
        mesh=mesh,
        scratch_types=[
            pltpu.VMEM((MAXSLABS * SLAB,), jnp.int32),
            pltpu.VMEM((MAXSLABS * SLAB,), jnp.int32),
            pltpu.VMEM((MAXSLABS * SLAB,), jnp.int32),
            pltpu.VMEM((MAXSLABS * SLAB,), jnp.int32),
            pltpu.VMEM((K, SLAB, EMB), jnp.float32),
            pltpu.VMEM((V0, EMB), jnp.float32),
            pltpu.VMEM((V1, EMB), jnp.float32),
            pltpu.VMEM((V2, EMB), jnp.float32),
            pltpu.VMEM((NCOMBO, EMB), jnp.float32),
            pltpu.SemaphoreType.DMA((K,)),
        ],
    )(e0, e1, e2, w0, w1, w2)
    return out


def kernel(edge_attr, W0, W1, W2):
    ea = edge_attr.astype(jnp.int32)
    return _run(ea[:, 0], ea[:, 1], ea[:, 2], W0, W1, W2)


# speedup vs baseline: 11.7368x; 2.3164x over previous
import jax
import jax.numpy as jnp
from jax import lax
from jax.experimental import pallas as pl
from jax.experimental.pallas import tpu as pltpu
from jax.experimental.pallas import tpu_sc as plsc

EMB = 128
V0, V1, V2 = 4, 2, 6
NCOMBO = V0 * V1 * V2
E = 320000
NC, NS = 2, 16
NW = NC * NS
SLAB = 128
NSLAB = E // SLAB
BASE = NSLAB // NW
EXTRA = NSLAB % NW
MAXSLABS = BASE + 1
K = 4


def _bond_kernel(e0_hbm, e1_hbm, e2_hbm, w0_hbm, w1_hbm, w2_hbm, out_hbm,
                 e0_v, e1_v, e2_v, pk_v, rows_v, w0_v, w1_v, w2_v,
                 t_v, ssem):
    cid = lax.axis_index("c")
    sid = lax.axis_index("s")
    wid = sid * NC + cid

    pltpu.sync_copy(w0_hbm, w0_v)
    pltpu.sync_copy(w1_hbm, w1_v)
    pltpu.sync_copy(w2_hbm, w2_v)
    for r in range(NCOMBO):
        i, j, k = r // (V1 * V2), (r // V2) % V1, r % V2

        def _g(g, carry, i=i, j=j, k=k, r=r):
            sl = pl.ds(g * 16, 16)
            t_v[r, sl] = w0_v[i, sl] + w1_v[j, sl] + w2_v[k, sl]
            return carry

        lax.fori_loop(0, EMB // 16, _g, 0)

    start_slab = wid * BASE + jnp.maximum(wid - (NW - EXTRA), 0)
    ea_base = start_slab * SLAB
    pltpu.sync_copy(e0_hbm.at[pl.ds(ea_base, MAXSLABS * SLAB)], e0_v)
    pltpu.sync_copy(e1_hbm.at[pl.ds(ea_base, MAXSLABS * SLAB)], e1_v)
    pltpu.sync_copy(e2_hbm.at[pl.ds(ea_base, MAXSLABS * SLAB)], e2_v)

    def _pk(t, carry):
        sl = pl.ds(t * 16, 16)
        pk_v[sl] = e0_v[sl] * (V1 * V2) + e1_v[sl] * V2 + e2_v[sl]
        return carry

    lax.fori_loop(0, MAXSLABS * SLAB // 16, _pk, 0)

    nslabs = jnp.where(wid >= NW - EXTRA, MAXSLABS, BASE)

    def _swait(slot):
        pltpu.make_async_copy(rows_v.at[slot], out_hbm.at[pl.ds(0, SLAB)],
                              ssem.at[slot]).wait()

    def _slab(t, carry):
        slot = lax.rem(t, K)

        @pl.when(t >= K)
        def _buffer_free():
            _swait(slot)

        def _grp(q, c2):
            pk16 = pk_v[pl.ds(t * SLAB + q * 16, 16)]

            def _row(li):
                r = pk16[li]
                return [t_v[r, pl.ds(g * 16, 16)] for g in range(EMB // 16)]

            prev = _row(0)
            for li in range(1, 16):
                cur = _row(li)
                e = q * 16 + li - 1
                for g in range(EMB // 16):
                    rows_v[slot, e, pl.ds(g * 16, 16)] = prev[g]
                prev = cur
            e = q * 16 + 15
            for g in range(EMB // 16):
                rows_v[slot, e, pl.ds(g * 16, 16)] = prev[g]
            return c2

        lax.fori_loop(0, SLAB // 16, _grp, 0)
        pltpu.async_copy(rows_v.at[slot],
                         out_hbm.at[pl.ds((start_slab + t) * SLAB, SLAB)],
                         ssem.at[slot])
        return carry

    lax.fori_loop(0, nslabs, _slab, 0)
    for s in range(K):
        _swait(s)


@jax.jit
def _run(e0, e1, e2, w0, w1, w2):
    mesh = plsc.VectorSubcoreMesh(core_axis_name="c", subcore_axis_name="s",
                                  num_cores=NC, num_subcores=NS)
    out = pl.kernel(
        _bond_kernel,
        out_type=jax.ShapeDtypeStruct((E, EMB), jnp.float32),
        mesh=mesh,
        scratch_types=[
            pltpu.VMEM((MAXSLABS * SLAB,), jnp.int32),
            pltpu.VMEM((MAXSLABS * SLAB,), jnp.int32),
            pltpu.VMEM((MAXSLABS * SLAB,), jnp.int32),
            pltpu.VMEM((MAXSLABS * SLAB,), jnp.int32),
            pltpu.VMEM((K, SLAB, EMB), jnp.float32),
            pltpu.VMEM((V0, EMB), jnp.float32),
            pltpu.VMEM((V1, EMB), jnp.float32),
            pltpu.VMEM((V2, EMB), jnp.float32),
            pltpu.VMEM((NCOMBO, EMB), jnp.float32),
            pltpu.SemaphoreType.DMA((K,)),
        ],
    )(e0, e1, e2, w0, w1, w2)
    return out


def kernel(edge_attr, W0, W1, W2):
    ea = edge_attr.astype(jnp.int32)
    return _run(ea[:, 0], ea[:, 1], ea[:, 2], W0, W1, W2)
